# baseline (speedup 1.0000x reference)
import functools

import jax
import jax.numpy as jnp
from jax import lax
from jax.experimental import pallas as pl
from jax.experimental.pallas import tpu as pltpu
from jax.experimental.pallas import tpu_sc as plsc

NC = 2
NS = 16
NW = NC * NS
L = 16

D = 128
WEXT = 144
GB = 64
NBUF = 3
CB2 = 64
CB3 = 48
AR = 10112
ERR = NW * GB * 5


def _prep_body(prel_ref, pgrel_ref, pgdst_ref, pgori_ref, relcomp_ref,
               relfeat_ref, relhead_ref, reltail_ref, wrel_ref,
               timefeat_ref, wtime_ref,
               v2_ref, initrel_ref, relemb_ref, timeemb_ref):
    f32 = jnp.float32
    prel = prel_ref[...]
    pgr = pgrel_ref[...]
    pgd = pgdst_ref[...]
    pgo = pgori_ref[...]
    epg = pgr.shape[0]
    npg = pgo.shape[0]
    nrelk = prel.shape[0]
    oh_rel = (pgr == lax.broadcasted_iota(jnp.int32, (epg, nrelk), 1)
              ).astype(f32)
    adst = (lax.broadcasted_iota(jnp.int32, (npg, epg), 0) == pgd
            ).astype(f32)
    s = jnp.dot(adst, oh_rel, preferred_element_type=f32)
    degp = jnp.sum(s, axis=1, keepdims=True)
    rpg = jnp.dot(s, prel, preferred_element_type=f32) / jnp.maximum(degp, 1.0)
    obs = pgo >= 0
    safe = jnp.where(obs, pgo, 0)
    nrel = relcomp_ref.shape[0]
    ohc = (safe == lax.broadcasted_iota(jnp.int32, (npg, nrel), 1)
           ).astype(f32)
    comp = jnp.dot(ohc, relcomp_ref[...], preferred_element_type=f32)
    rel_coef = jnp.where(obs, comp, rpg)
    heads = jnp.dot(rel_coef, relhead_ref[...], preferred_element_type=f32)
    tails = jnp.dot(rel_coef, reltail_ref[...], preferred_element_type=f32)
    init_rel = jnp.dot(rel_coef, relfeat_ref[...], preferred_element_type=f32)
    initrel_ref[...] = init_rel
    relemb_ref[...] = jnp.maximum(
        jnp.dot(init_rel, wrel_ref[...], preferred_element_type=f32), 0.0)
    timeemb_ref[...] = jnp.maximum(
        jnp.dot(timefeat_ref[...], wtime_ref[...], preferred_element_type=f32),
        0.0)
    both = jnp.concatenate([tails, heads], axis=0)
    ext = (lax.broadcasted_iota(jnp.int32, (2 * npg, WEXT - D), 1) == 0
           ).astype(f32)
    v2_ref[...] = jnp.concatenate([both, ext], axis=1)


def _k2_body(nch,
             v2_hbm, idx3_hbm, ori_hbm, ent_hbm, zero_hbm,
             hp_hbm, er_hbm,
             accum, rows0, rows1, rows2, erbuf, ib0, ib1, ib2,
             cb0, cb1, cb2, db0, db1, db2,
             oribuf, safebuf,
             semi0, semi1, semi2, semg0, semg1, semg2,
             sems0, sems1, sems2, seme):
    cid = lax.axis_index("c")
    sid = lax.axis_index("s")
    wid = cid * NS + sid
    rpt = AR // NS
    npg = v2_hbm.shape[0] // 2
    chbase = wid * nch
    ib = (ib0, ib1, ib2)
    cb = (cb0, cb1, cb2)
    db = (db0, db1, db2)
    rows = (rows0, rows1, rows2)
    semi = (semi0, semi1, semi2)
    semg = (semg0, semg1, semg2)
    sems = (sems0, sems1, sems2)

    pltpu.sync_copy(zero_hbm.at[pl.ds(sid * rpt, rpt)],
                    accum.at[pl.ds(sid * rpt, rpt)])

    r_t = ERR // NW
    rbase = wid * r_t

    @pl.loop(0, r_t // GB)
    def _gath(k):
        off = rbase + k * GB
        pltpu.sync_copy(ori_hbm.at[pl.ds(off, GB)], oribuf)
        for j in range(GB // L):
            v = oribuf[pl.ds(j * L, L)]
            safebuf[pl.ds(j * L, L)] = jnp.maximum(v, 0)
        pltpu.async_copy(ent_hbm.at[safebuf], erbuf, seme).wait()
        pltpu.sync_copy(erbuf, er_hbm.at[pl.ds(off, GB)])

    plsc.subcore_barrier()

    def start_idx(b, ch):
        pltpu.async_copy(
            idx3_hbm.at[pl.ds((chbase + ch) * (3 * CB2), 3 * CB2)],
            ib[b], semi[b])

    def wait_idx(b):
        pltpu.make_async_copy(idx3_hbm.at[pl.ds(0, 3 * CB2)], ib[b],
                              semi[b]).wait()

    def wait_scat(b):
        pltpu.make_async_copy(rows[b], accum.at[db[b]], sems[b]).wait()

    for b in range(NBUF):
        start_idx(b, b)

    @pl.loop(0, nch // NBUF)
    def _main(jj):
        descs = []
        for b in range(NBUF):
            wait_idx(b)

            @pl.when(jj >= 1)
            def _(b=b):
                wait_scat(b)

            for j in range(CB2 // L):
                brelv = ib[b][pl.ds(j * L, L)]
                invv = ib[b][pl.ds(CB2 + j * L, L)]
                cb[b][pl.ds(j * L, L)] = brelv + invv * npg
                db[b][pl.ds(j * L, L)] = ib[b][pl.ds(2 * CB2 + j * L, L)]
            descs.append(pltpu.async_copy(v2_hbm.at[cb[b]], rows[b], semg[b]))
        for b in range(NBUF):
            descs[b].wait()

            @pl.when(jj < nch // NBUF - 1)
            def _(b=b):
                start_idx(b, NBUF * jj + NBUF + b)

            pltpu.async_copy(rows[b], accum.at[db[b]], sems[b], add=True)

    for b in range(NBUF):
        wait_scat(b)
    plsc.subcore_barrier()

    pltpu.sync_copy(
        accum.at[pl.ds(sid * rpt, rpt)],
        hp_hbm.at[pl.ds(cid * AR + sid * rpt, rpt)])


def _combine_body(hp0_ref, hp1_ref, er_ref, ori_ref, ie_ref, deg_ref):
    s = hp0_ref[...] + hp1_ref[...]
    deg = s[:, D:D + 1]
    h = s[:, :D] / jnp.maximum(deg, 1.0)
    obs = ori_ref[...] >= 0
    ie_ref[...] = jnp.where(obs, er_ref[...], h)
    deg_ref[...] = deg


def _k3_body(nch,
             ie_hbm, ir_hbm, idx3_hbm, zero_hbm,
             aggp_hbm,
             accum, irtab, ib0, ib1, ib2, db0, db1, db2,
             ieb0, ieb1, ieb2,
             semi0, semi1, semi2, semg0, semg1, semg2,
             sems0, sems1, sems2):
    cid = lax.axis_index("c")
    sid = lax.axis_index("s")
    wid = cid * NS + sid
    rpt = AR // NS
    chbase = wid * nch
    ib = (ib0, ib1, ib2)
    db = (db0, db1, db2)
    ieb = (ieb0, ieb1, ieb2)
    semi = (semi0, semi1, semi2)
    semg = (semg0, semg1, semg2)
    sems = (sems0, sems1, sems2)

    pltpu.sync_copy(zero_hbm.at[pl.ds(sid * rpt, rpt)],
                    accum.at[pl.ds(sid * rpt, rpt)])
    pltpu.sync_copy(ir_hbm, irtab)
    plsc.subcore_barrier()

    def start_idx(b, ch):
        pltpu.async_copy(
            idx3_hbm.at[pl.ds((chbase + ch) * (3 * CB3), 3 * CB3)],
            ib[b], semi[b])

    def wait_idx(b):
        pltpu.make_async_copy(idx3_hbm.at[pl.ds(0, 3 * CB3)], ib[b],
                              semi[b]).wait()

    def wait_scat(b):
        pltpu.make_async_copy(ieb[b], accum.at[db[b]], sems[b]).wait()

    for b in range(NBUF):
        start_idx(b, b)

    @pl.loop(0, nch // NBUF)
    def _main(jj):
        descs = []
        for b in range(NBUF):
            wait_idx(b)

            @pl.when(jj >= 1)
            def _(b=b):
                wait_scat(b)

            for j in range(CB3 // L):
                db[b][pl.ds(j * L, L)] = ib[b][pl.ds(2 * CB3 + j * L, L)]
            descs.append(pltpu.async_copy(ie_hbm.at[ib[b].at[pl.ds(0, CB3)]],
                                          ieb[b], semg[b]))
        for b in range(NBUF):
            descs[b].wait()

            @pl.when(jj < nch // NBUF - 1)
            def _(b=b):
                start_idx(b, NBUF * jj + NBUF + b)

            @plsc.parallel_loop(0, CB3 // L)
            def _mul(g):
                bvec = ib[b][pl.ds(CB3 + g * L, L)]
                for lane in range(L):
                    r = g * L + lane
                    bv = bvec[lane]
                    for j in range(D // L):
                        ieb[b][r, pl.ds(j * L, L)] = (
                            ieb[b][r, pl.ds(j * L, L)]
                            * irtab[bv, pl.ds(j * L, L)])

            pltpu.async_copy(ieb[b], accum.at[db[b]], sems[b], add=True)

    for b in range(NBUF):
        wait_scat(b)
    plsc.subcore_barrier()

    pltpu.sync_copy(
        accum.at[pl.ds(sid * rpt, rpt)],
        aggp_hbm.at[pl.ds(cid * AR + sid * rpt, rpt)])


def _final_body(a0_ref, a1_ref, deg_ref, ie_ref, went_ref, wself_ref, out_ref):
    f32 = jnp.float32
    a = (a0_ref[...] + a1_ref[...]) / jnp.maximum(deg_ref[...], 1.0)
    out_ref[...] = jnp.maximum(
        jnp.dot(a, went_ref[...], preferred_element_type=f32)
        + jnp.dot(ie_ref[...], wself_ref[...], preferred_element_type=f32),
        0.0)


def kernel(rel_comp, rel_feat, ent_feat, rel_head_feat, rel_tail_feat,
           pattern_rel_ent, time_feat, W_ent, W_self, W_rel, W_time,
           g_edge_index, g_b_rel, g_inv, g_ori_idx,
           pg_edge_index, pg_rel, pg_ori_idx):
    f32 = jnp.float32
    i32 = jnp.int32
    n = g_ori_idx.shape[0]
    e = g_b_rel.shape[0]
    npg = pg_ori_idx.shape[0]
    epg = pg_rel.shape[0]
    ntime = time_feat.shape[0]

    v2ext, init_rel, rel_emb, time_emb = pl.pallas_call(
        _prep_body,
        out_shape=[
            jax.ShapeDtypeStruct((2 * npg, WEXT), f32),
            jax.ShapeDtypeStruct((npg, D), f32),
            jax.ShapeDtypeStruct((npg, D), f32),
            jax.ShapeDtypeStruct((ntime, D), f32),
        ],
    )(pattern_rel_ent,
      pg_rel.astype(i32).reshape(epg, 1),
      pg_edge_index[1].astype(i32).reshape(1, epg),
      pg_ori_idx.astype(i32).reshape(npg, 1),
      rel_comp, rel_feat, rel_head_feat, rel_tail_feat, W_rel,
      time_feat, W_time)

    assert n <= AR - 1

    def _pad_edges(arrs, cbn):
        unit = NBUF * NW * cbn
        epad = ((e + unit - 1) // unit) * unit
        padded = []
        for a, fill in arrs:
            padded.append(jnp.pad(a.astype(i32), (0, epad - e),
                                  constant_values=fill))
        nch = epad // (NW * cbn)
        idx3 = jnp.stack([p.reshape(-1, cbn) for p in padded],
                         axis=1).reshape(-1)
        return idx3, nch

    src = g_edge_index[0]
    dst = g_edge_index[1]
    idx3_k2, nch2 = _pad_edges([(g_b_rel, 0), (g_inv, 0), (dst, n)], CB2)
    idx3_k3, nch3 = _pad_edges([(src, 0), (g_b_rel, 0), (dst, n)], CB3)
    ori = jnp.pad(g_ori_idx.astype(i32), (0, ERR - n), constant_values=-1)
    zeros_w = jnp.zeros((AR, WEXT), f32)
    zeros_d = jnp.zeros((AR, D), f32)

    mesh = plsc.VectorSubcoreMesh(core_axis_name="c", subcore_axis_name="s",
                                  num_cores=NC, num_subcores=NS)
    k2 = pl.kernel(
        functools.partial(_k2_body, nch2),
        out_type=[
            jax.ShapeDtypeStruct((NC * AR, WEXT), f32),
            jax.ShapeDtypeStruct((ERR, D), f32),
        ],
        mesh=mesh,
        compiler_params=pltpu.CompilerParams(use_tc_tiling_on_sc=False),
        scratch_types=(
            [pltpu.VMEM_SHARED((AR, WEXT), f32)]
            + [pltpu.VMEM((CB2, WEXT), f32)] * NBUF
            + [pltpu.VMEM((GB, D), f32)]
            + [pltpu.VMEM((3 * CB2,), i32)] * NBUF
            + [pltpu.VMEM((CB2,), i32)] * NBUF
            + [pltpu.VMEM((CB2,), i32)] * NBUF
            + [pltpu.VMEM((GB,), i32)] * 2
            + [pltpu.SemaphoreType.DMA] * (3 * NBUF + 1)
        ),
    )
    hp, er = k2(v2ext, idx3_k2, ori, ent_feat, zeros_w)

    br = AR // 16
    init_ent, deg = pl.pallas_call(
        _combine_body,
        grid=(16,),
        in_specs=[
            pl.BlockSpec((br, WEXT), lambda i: (i, 0)),
            pl.BlockSpec((br, WEXT), lambda i: (i, 0)),
            pl.BlockSpec((br, D), lambda i: (i, 0)),
            pl.BlockSpec((br, 1), lambda i: (i, 0)),
        ],
        out_specs=[
            pl.BlockSpec((br, D), lambda i: (i, 0)),
            pl.BlockSpec((br, 1), lambda i: (i, 0)),
        ],
        out_shape=[
            jax.ShapeDtypeStruct((AR, D), f32),
            jax.ShapeDtypeStruct((AR, 1), f32),
        ],
    )(hp[:AR], hp[AR:], er[:AR], ori[:AR].reshape(AR, 1))

    k3 = pl.kernel(
        functools.partial(_k3_body, nch3),
        out_type=jax.ShapeDtypeStruct((NC * AR, D), f32),
        mesh=mesh,
        compiler_params=pltpu.CompilerParams(use_tc_tiling_on_sc=False),
        scratch_types=(
            [pltpu.VMEM_SHARED((AR, D), f32)]
            + [pltpu.VMEM((npg, D), f32)]
            + [pltpu.VMEM((3 * CB3,), i32)] * NBUF
            + [pltpu.VMEM((CB3,), i32)] * NBUF
            + [pltpu.VMEM((CB3, D), f32)] * NBUF
            + [pltpu.SemaphoreType.DMA] * (3 * NBUF)
        ),
    )
    aggp = k3(init_ent, init_rel, idx3_k3, zeros_d)

    ent_full = pl.pallas_call(
        _final_body,
        grid=(16,),
        in_specs=[
            pl.BlockSpec((br, D), lambda i: (i, 0)),
            pl.BlockSpec((br, D), lambda i: (i, 0)),
            pl.BlockSpec((br, 1), lambda i: (i, 0)),
            pl.BlockSpec((br, D), lambda i: (i, 0)),
            pl.BlockSpec((D, D), lambda i: (0, 0)),
            pl.BlockSpec((D, D), lambda i: (0, 0)),
        ],
        out_specs=pl.BlockSpec((br, D), lambda i: (i, 0)),
        out_shape=jax.ShapeDtypeStruct((AR, D), f32),
    )(aggp[:AR], aggp[AR:], deg, init_ent, W_ent, W_self)

    return (ent_full[:n], rel_emb, time_emb)


# speedup vs baseline: 1.4019x; 1.4019x over previous
import functools

import jax
import jax.numpy as jnp
from jax import lax
from jax.experimental import pallas as pl
from jax.experimental.pallas import tpu as pltpu
from jax.experimental.pallas import tpu_sc as plsc

NC = 2
NS = 16
NW = NC * NS
L = 16

D = 128
WEXT = 144
GB = 64
NBUF = 3
CB2 = 64
CB3 = 48
AR = 10112
ERR = NW * GB * 5


def _prep_body(prel_ref, pgrel_ref, pgdst_ref, pgori_ref, relcomp_ref,
               relfeat_ref, relhead_ref, reltail_ref, wrel_ref,
               timefeat_ref, wtime_ref,
               v2_ref, initrel_ref, relemb_ref, timeemb_ref):
    f32 = jnp.float32
    prel = prel_ref[...]
    pgr = pgrel_ref[...]
    pgd = pgdst_ref[...]
    pgo = pgori_ref[...]
    epg = pgr.shape[0]
    npg = pgo.shape[0]
    nrelk = prel.shape[0]
    oh_rel = (pgr == lax.broadcasted_iota(jnp.int32, (epg, nrelk), 1)
              ).astype(f32)
    adst = (lax.broadcasted_iota(jnp.int32, (npg, epg), 0) == pgd
            ).astype(f32)
    s = jnp.dot(adst, oh_rel, preferred_element_type=f32)
    degp = jnp.sum(s, axis=1, keepdims=True)
    rpg = jnp.dot(s, prel, preferred_element_type=f32) / jnp.maximum(degp, 1.0)
    obs = pgo >= 0
    safe = jnp.where(obs, pgo, 0)
    nrel = relcomp_ref.shape[0]
    ohc = (safe == lax.broadcasted_iota(jnp.int32, (npg, nrel), 1)
           ).astype(f32)
    comp = jnp.dot(ohc, relcomp_ref[...], preferred_element_type=f32)
    rel_coef = jnp.where(obs, comp, rpg)
    heads = jnp.dot(rel_coef, relhead_ref[...], preferred_element_type=f32)
    tails = jnp.dot(rel_coef, reltail_ref[...], preferred_element_type=f32)
    init_rel = jnp.dot(rel_coef, relfeat_ref[...], preferred_element_type=f32)
    initrel_ref[...] = init_rel
    relemb_ref[...] = jnp.maximum(
        jnp.dot(init_rel, wrel_ref[...], preferred_element_type=f32), 0.0)
    timeemb_ref[...] = jnp.maximum(
        jnp.dot(timefeat_ref[...], wtime_ref[...], preferred_element_type=f32),
        0.0)
    both = jnp.concatenate([tails, heads], axis=0)
    ext = (lax.broadcasted_iota(jnp.int32, (2 * npg, WEXT - D), 1) == 0
           ).astype(f32)
    v2_ref[...] = jnp.concatenate([both, ext], axis=1)


def _k2_body(nch,
             v2_hbm, idx3_hbm, ori_hbm, ent_hbm, zero_hbm,
             hp_hbm, er_hbm,
             accum, rows0, rows1, rows2, erbuf, ib0, ib1, ib2,
             cb0, cb1, cb2, db0, db1, db2,
             oribuf, safebuf,
             semi0, semi1, semi2, semg0, semg1, semg2,
             sems0, sems1, sems2, seme):
    cid = lax.axis_index("c")
    sid = lax.axis_index("s")
    wid = cid * NS + sid
    rpt = AR // NS
    npg = v2_hbm.shape[0] // 2
    chbase = wid * nch
    ib = (ib0, ib1, ib2)
    cb = (cb0, cb1, cb2)
    db = (db0, db1, db2)
    rows = (rows0, rows1, rows2)
    semi = (semi0, semi1, semi2)
    semg = (semg0, semg1, semg2)
    sems = (sems0, sems1, sems2)

    pltpu.sync_copy(zero_hbm.at[pl.ds(sid * rpt, rpt)],
                    accum.at[pl.ds(sid * rpt, rpt)])

    r_t = ERR // NW
    rbase = wid * r_t

    @pl.loop(0, r_t // GB)
    def _gath(k):
        off = rbase + k * GB
        pltpu.sync_copy(ori_hbm.at[pl.ds(off, GB)], oribuf)
        for j in range(GB // L):
            v = oribuf[pl.ds(j * L, L)]
            safebuf[pl.ds(j * L, L)] = jnp.maximum(v, 0)
        pltpu.async_copy(ent_hbm.at[safebuf], erbuf, seme).wait()
        pltpu.sync_copy(erbuf, er_hbm.at[pl.ds(off, GB)])

    plsc.subcore_barrier()

    def start_idx(b, ch):
        pltpu.async_copy(
            idx3_hbm.at[pl.ds((chbase + ch) * (3 * CB2), 3 * CB2)],
            ib[b], semi[b])

    def wait_idx(b):
        pltpu.make_async_copy(idx3_hbm.at[pl.ds(0, 3 * CB2)], ib[b],
                              semi[b]).wait()

    def wait_scat(b):
        pltpu.make_async_copy(rows[b], accum.at[db[b]], sems[b]).wait()

    for b in range(NBUF):
        start_idx(b, b)

    @pl.loop(0, nch // NBUF)
    def _main(jj):
        descs = []
        for b in range(NBUF):
            wait_idx(b)

            @pl.when(jj >= 1)
            def _(b=b):
                wait_scat(b)

            for j in range(CB2 // L):
                brelv = ib[b][pl.ds(j * L, L)]
                invv = ib[b][pl.ds(CB2 + j * L, L)]
                cb[b][pl.ds(j * L, L)] = brelv + invv * npg
                db[b][pl.ds(j * L, L)] = ib[b][pl.ds(2 * CB2 + j * L, L)]
            descs.append(pltpu.async_copy(v2_hbm.at[cb[b]], rows[b], semg[b]))
        for b in range(NBUF):
            descs[b].wait()

            @pl.when(jj < nch // NBUF - 1)
            def _(b=b):
                start_idx(b, NBUF * jj + NBUF + b)

            pltpu.async_copy(rows[b], accum.at[db[b]], sems[b], add=True)

    for b in range(NBUF):
        wait_scat(b)
    plsc.subcore_barrier()

    pltpu.sync_copy(
        accum.at[pl.ds(sid * rpt, rpt)],
        hp_hbm.at[pl.ds(cid * AR + sid * rpt, rpt)])


def _combine_body(hp0_ref, hp1_ref, er_ref, ori_ref, ie_ref, deg_ref):
    s = hp0_ref[...] + hp1_ref[...]
    deg = s[:, D:D + 1]
    h = s[:, :D] / jnp.maximum(deg, 1.0)
    obs = ori_ref[...] >= 0
    ie_ref[...] = jnp.where(obs, er_ref[...], h)
    deg_ref[...] = deg


def _k3_body(nch,
             ie_hbm, ir_hbm, idx3_hbm, zero_hbm,
             aggp_hbm,
             accum, ib0, ib1, ib2, db0, db1, db2,
             ieb0, ieb1, ieb2, irb0, irb1, irb2, msg0, msg1, msg2,
             semi0, semi1, semi2, semg0, semg1, semg2,
             sems0, sems1, sems2):
    cid = lax.axis_index("c")
    sid = lax.axis_index("s")
    wid = cid * NS + sid
    rpt = AR // NS
    chbase = wid * nch
    ib = (ib0, ib1, ib2)
    db = (db0, db1, db2)
    ieb = (ieb0, ieb1, ieb2)
    irb = (irb0, irb1, irb2)
    msg = (msg0, msg1, msg2)
    semi = (semi0, semi1, semi2)
    semg = (semg0, semg1, semg2)
    sems = (sems0, sems1, sems2)

    pltpu.sync_copy(zero_hbm.at[pl.ds(sid * rpt, rpt)],
                    accum.at[pl.ds(sid * rpt, rpt)])
    plsc.subcore_barrier()

    def start_idx(b, ch):
        pltpu.async_copy(
            idx3_hbm.at[pl.ds((chbase + ch) * (3 * CB3), 3 * CB3)],
            ib[b], semi[b])

    def wait_idx(b):
        pltpu.make_async_copy(idx3_hbm.at[pl.ds(0, 3 * CB3)], ib[b],
                              semi[b]).wait()

    def wait_scat(b):
        pltpu.make_async_copy(msg[b], accum.at[db[b]], sems[b]).wait()

    for b in range(NBUF):
        start_idx(b, b)

    @pl.loop(0, nch // NBUF)
    def _main(jj):
        descs = []
        for b in range(NBUF):
            wait_idx(b)

            @pl.when(jj >= 1)
            def _(b=b):
                wait_scat(b)

            for j in range(CB3 // L):
                db[b][pl.ds(j * L, L)] = ib[b][pl.ds(2 * CB3 + j * L, L)]
            ga = pltpu.async_copy(ie_hbm.at[ib[b].at[pl.ds(0, CB3)]],
                                  ieb[b], semg[b])
            gb = pltpu.async_copy(ir_hbm.at[ib[b].at[pl.ds(CB3, CB3)]],
                                  irb[b], semg[b])
            descs.append((ga, gb))
        for b in range(NBUF):
            ga, gb = descs[b]
            ga.wait()
            gb.wait()

            @pl.when(jj < nch // NBUF - 1)
            def _(b=b):
                start_idx(b, NBUF * jj + NBUF + b)

            @plsc.parallel_loop(0, CB3)
            def _mul(r):
                for j in range(D // (2 * L)):
                    va = ieb[b][r, pl.ds(2 * L * j, 2 * L)]
                    vb = irb[b][r, pl.ds(2 * L * j, 2 * L)]
                    a0, a1 = plsc.unpack(va, format=plsc.PackFormat.INTERLEAVED)
                    b0, b1 = plsc.unpack(vb, format=plsc.PackFormat.INTERLEAVED)
                    msg[b][r, pl.ds(2 * L * j, L)] = a0 * b0
                    msg[b][r, pl.ds(2 * L * j + L, L)] = a1 * b1

            pltpu.async_copy(msg[b], accum.at[db[b]], sems[b], add=True)

    for b in range(NBUF):
        wait_scat(b)
    plsc.subcore_barrier()

    pltpu.sync_copy(
        accum.at[pl.ds(sid * rpt, rpt)],
        aggp_hbm.at[pl.ds(cid * AR + sid * rpt, rpt)])


def _final_body(a0_ref, a1_ref, deg_ref, ie_ref, went_ref, wself_ref, out_ref):
    f32 = jnp.float32
    a = (a0_ref[...] + a1_ref[...]) / jnp.maximum(deg_ref[...], 1.0)
    out_ref[...] = jnp.maximum(
        jnp.dot(a, went_ref[...], preferred_element_type=f32)
        + jnp.dot(ie_ref[...], wself_ref[...], preferred_element_type=f32),
        0.0)


def kernel(rel_comp, rel_feat, ent_feat, rel_head_feat, rel_tail_feat,
           pattern_rel_ent, time_feat, W_ent, W_self, W_rel, W_time,
           g_edge_index, g_b_rel, g_inv, g_ori_idx,
           pg_edge_index, pg_rel, pg_ori_idx):
    f32 = jnp.float32
    i32 = jnp.int32
    n = g_ori_idx.shape[0]
    e = g_b_rel.shape[0]
    npg = pg_ori_idx.shape[0]
    epg = pg_rel.shape[0]
    ntime = time_feat.shape[0]

    v2ext, init_rel, rel_emb, time_emb = pl.pallas_call(
        _prep_body,
        out_shape=[
            jax.ShapeDtypeStruct((2 * npg, WEXT), f32),
            jax.ShapeDtypeStruct((npg, D), f32),
            jax.ShapeDtypeStruct((npg, D), f32),
            jax.ShapeDtypeStruct((ntime, D), f32),
        ],
    )(pattern_rel_ent,
      pg_rel.astype(i32).reshape(epg, 1),
      pg_edge_index[1].astype(i32).reshape(1, epg),
      pg_ori_idx.astype(i32).reshape(npg, 1),
      rel_comp, rel_feat, rel_head_feat, rel_tail_feat, W_rel,
      time_feat, W_time)

    assert n <= AR - 1

    def _pad_edges(arrs, cbn):
        unit = NBUF * NW * cbn
        epad = ((e + unit - 1) // unit) * unit
        padded = []
        for a, fill in arrs:
            padded.append(jnp.pad(a.astype(i32), (0, epad - e),
                                  constant_values=fill))
        nch = epad // (NW * cbn)
        idx3 = jnp.stack([p.reshape(-1, cbn) for p in padded],
                         axis=1).reshape(-1)
        return idx3, nch

    src = g_edge_index[0]
    dst = g_edge_index[1]
    idx3_k2, nch2 = _pad_edges([(g_b_rel, 0), (g_inv, 0), (dst, n)], CB2)
    idx3_k3, nch3 = _pad_edges([(src, 0), (g_b_rel, 0), (dst, n)], CB3)
    ori = jnp.pad(g_ori_idx.astype(i32), (0, ERR - n), constant_values=-1)
    zeros_w = jnp.zeros((AR, WEXT), f32)
    zeros_d = jnp.zeros((AR, D), f32)

    mesh = plsc.VectorSubcoreMesh(core_axis_name="c", subcore_axis_name="s",
                                  num_cores=NC, num_subcores=NS)
    k2 = pl.kernel(
        functools.partial(_k2_body, nch2),
        out_type=[
            jax.ShapeDtypeStruct((NC * AR, WEXT), f32),
            jax.ShapeDtypeStruct((ERR, D), f32),
        ],
        mesh=mesh,
        compiler_params=pltpu.CompilerParams(use_tc_tiling_on_sc=False),
        scratch_types=(
            [pltpu.VMEM_SHARED((AR, WEXT), f32)]
            + [pltpu.VMEM((CB2, WEXT), f32)] * NBUF
            + [pltpu.VMEM((GB, D), f32)]
            + [pltpu.VMEM((3 * CB2,), i32)] * NBUF
            + [pltpu.VMEM((CB2,), i32)] * NBUF
            + [pltpu.VMEM((CB2,), i32)] * NBUF
            + [pltpu.VMEM((GB,), i32)] * 2
            + [pltpu.SemaphoreType.DMA] * (3 * NBUF + 1)
        ),
    )
    hp, er = k2(v2ext, idx3_k2, ori, ent_feat, zeros_w)

    br = AR // 16
    init_ent, deg = pl.pallas_call(
        _combine_body,
        grid=(16,),
        in_specs=[
            pl.BlockSpec((br, WEXT), lambda i: (i, 0)),
            pl.BlockSpec((br, WEXT), lambda i: (i, 0)),
            pl.BlockSpec((br, D), lambda i: (i, 0)),
            pl.BlockSpec((br, 1), lambda i: (i, 0)),
        ],
        out_specs=[
            pl.BlockSpec((br, D), lambda i: (i, 0)),
            pl.BlockSpec((br, 1), lambda i: (i, 0)),
        ],
        out_shape=[
            jax.ShapeDtypeStruct((AR, D), f32),
            jax.ShapeDtypeStruct((AR, 1), f32),
        ],
    )(hp[:AR], hp[AR:], er[:AR], ori[:AR].reshape(AR, 1))

    cols = jnp.arange(D)
    perm = (cols // 32) * 32 + (cols % 2) * 16 + (cols % 32) // 2
    ie_b16 = init_ent[:, perm].astype(jnp.bfloat16)
    ir_b16 = init_rel[:, perm].astype(jnp.bfloat16)

    k3 = pl.kernel(
        functools.partial(_k3_body, nch3),
        out_type=jax.ShapeDtypeStruct((NC * AR, D), f32),
        mesh=mesh,
        compiler_params=pltpu.CompilerParams(use_tc_tiling_on_sc=False,
                                             needs_layout_passes=False),
        scratch_types=(
            [pltpu.VMEM_SHARED((AR, D), f32)]
            + [pltpu.VMEM((3 * CB3,), i32)] * NBUF
            + [pltpu.VMEM((CB3,), i32)] * NBUF
            + [pltpu.VMEM((CB3, D), jnp.bfloat16)] * NBUF
            + [pltpu.VMEM((CB3, D), jnp.bfloat16)] * NBUF
            + [pltpu.VMEM((CB3, D), f32)] * NBUF
            + [pltpu.SemaphoreType.DMA] * (3 * NBUF)
        ),
    )
    aggp = k3(ie_b16, ir_b16, idx3_k3, zeros_d)

    ent_full = pl.pallas_call(
        _final_body,
        grid=(16,),
        in_specs=[
            pl.BlockSpec((br, D), lambda i: (i, 0)),
            pl.BlockSpec((br, D), lambda i: (i, 0)),
            pl.BlockSpec((br, 1), lambda i: (i, 0)),
            pl.BlockSpec((br, D), lambda i: (i, 0)),
            pl.BlockSpec((D, D), lambda i: (0, 0)),
            pl.BlockSpec((D, D), lambda i: (0, 0)),
        ],
        out_specs=pl.BlockSpec((br, D), lambda i: (i, 0)),
        out_shape=jax.ShapeDtypeStruct((AR, D), f32),
    )(aggp[:AR], aggp[AR:], deg, init_ent, W_ent, W_self)

    return (ent_full[:n], rel_emb, time_emb)
